# baseline (speedup 1.0000x reference)
import functools

import jax
import jax.numpy as jnp
from jax import lax
from jax.experimental import pallas as pl
from jax.experimental.pallas import tpu as pltpu
from jax.experimental.pallas import tpu_sc as plsc

N = 16384
C = 256
H = 1024
OUT = 512
BLK = 1024
NBLK = N // BLK
NW = 32
ROWS_W = N // NW
CH = 64
NCH = ROWS_W // CH


def _sc_mesh():
    return plsc.VectorSubcoreMesh(core_axis_name="c", subcore_axis_name="s")


CW = C // 2


def _pack_bf16_words(tab):
    v = lax.bitcast_convert_type(tab, jnp.uint32)
    even = (v[:, 0::2] + jnp.uint32(0x8000)) >> jnp.uint32(16)
    odd = (v[:, 1::2] + jnp.uint32(0x8000)) & jnp.uint32(0xFFFF0000)
    return lax.bitcast_convert_type(odd | even, jnp.int32)


def _sc_gather4(ia, if_, it, im, tab_a, tab_o, tab_m):
    @functools.partial(
        pl.kernel,
        mesh=_sc_mesh(),
        out_type=[jax.ShapeDtypeStruct((N, CW), jnp.int32) for _ in range(4)],
        scratch_types=(
            [pltpu.VMEM((CH,), jnp.int32) for _ in range(4)]
            + [pltpu.VMEM((CH, CW), jnp.int32) for _ in range(4)]
            + [pltpu.SemaphoreType.DMA for _ in range(4)]
        ),
    )
    def k(ia_h, if_h, it_h, im_h, ta_h, to_h, tm_h,
          oa_h, of_h, ot_h, om_h,
          iv0, iv1, iv2, iv3, rv0, rv1, rv2, rv3, s0, s1, s2, s3):
        wid = lax.axis_index("s") * 2 + lax.axis_index("c")
        base = wid * ROWS_W

        def chunk(c, carry):
            off = base + c * CH
            pltpu.sync_copy(ia_h.at[pl.ds(off, CH)], iv0)
            pltpu.sync_copy(if_h.at[pl.ds(off, CH)], iv1)
            pltpu.sync_copy(it_h.at[pl.ds(off, CH)], iv2)
            pltpu.sync_copy(im_h.at[pl.ds(off, CH)], iv3)
            c0 = pltpu.async_copy(ta_h.at[iv0], rv0, s0)
            c1 = pltpu.async_copy(to_h.at[iv1], rv1, s1)
            c2 = pltpu.async_copy(to_h.at[iv2], rv2, s2)
            c3 = pltpu.async_copy(tm_h.at[iv3], rv3, s3)
            c0.wait()
            c1.wait()
            c2.wait()
            c3.wait()
            pltpu.sync_copy(rv0, oa_h.at[pl.ds(off, CH)])
            pltpu.sync_copy(rv1, of_h.at[pl.ds(off, CH)])
            pltpu.sync_copy(rv2, ot_h.at[pl.ds(off, CH)])
            pltpu.sync_copy(rv3, om_h.at[pl.ds(off, CH)])
            return carry

        lax.fori_loop(0, NCH, chunk, 0)

    return k(ia, if_, it, im, tab_a, tab_o, tab_m)


def _sc_permute_rows(y, idx):
    @functools.partial(
        pl.kernel,
        mesh=_sc_mesh(),
        out_type=jax.ShapeDtypeStruct((N, OUT), jnp.float32),
        scratch_types=[
            pltpu.VMEM((CH,), jnp.int32),
            pltpu.VMEM((CH, OUT), jnp.float32),
            pltpu.SemaphoreType.DMA,
        ],
    )
    def k(y_h, idx_h, out_h, iv, rv, sem):
        wid = lax.axis_index("s") * 2 + lax.axis_index("c")
        base = wid * ROWS_W

        def chunk(c, carry):
            off = base + c * CH
            pltpu.sync_copy(idx_h.at[pl.ds(off, CH)], iv)
            pltpu.async_copy(y_h.at[iv], rv, sem).wait()
            pltpu.sync_copy(rv, out_h.at[pl.ds(off, CH)])
            return carry

        lax.fori_loop(0, NCH, chunk, 0)

    return k(y, idx)


def _tc_mlps(a, f, t, m, bounds, wait_row,
             pw1, pb1, pw2, pb2, tw1, tb1, tw2, tb2, mw1, mb1, mw2, mb2):
    def body(bounds_ref, a_ref, f_ref, t_ref, m_ref, wait_ref,
             pw1ae, pw1ao, pw1fe, pw1fo, pw1te, pw1to, pw1me, pw1mo,
             pb1_r, pw2_r, pb2_r,
             tw1ae, tw1ao, tw1me, tw1mo, tb1_r, tw2_r, tb2_r,
             mw1ae, mw1ao, mw1me, mw1mo, mb1_r, mw2_r, mb2_r,
             y_ref):
        g = pl.program_id(0)
        start = g * BLK
        end = start + BLK
        b1 = bounds_ref[0]
        b2 = bounds_ref[1]
        b3 = bounds_ref[2]
        row = start + lax.broadcasted_iota(jnp.int32, (BLK, 1), 0)

        y_ref[...] = jnp.broadcast_to(wait_ref[...], (BLK, OUT))

        def unpk(xw):
            u = lax.bitcast_convert_type(xw, jnp.uint32)
            ev = lax.bitcast_convert_type(u << jnp.uint32(16), jnp.float32)
            od = lax.bitcast_convert_type(u & jnp.uint32(0xFFFF0000), jnp.float32)
            return ev, od

        def mlp(parts, w1s, b1v, w2, b2v):
            acc = None
            for x, (we, wo) in zip(parts, w1s):
                ev, od = unpk(x)
                p = (jnp.dot(ev, we[...], preferred_element_type=jnp.float32)
                     + jnp.dot(od, wo[...], preferred_element_type=jnp.float32))
                acc = p if acc is None else acc + p
            hh = acc + b1v[...]
            hh = jnp.where(hh >= 0.0, hh, 0.01 * hh)
            return jnp.dot(hh, w2[...], preferred_element_type=jnp.float32) + b2v[...]

        @pl.when((start < b2) & (end > b1))
        def _():
            y = mlp([a_ref[...], f_ref[...], t_ref[...], m_ref[...]],
                    [(pw1ae, pw1ao), (pw1fe, pw1fo), (pw1te, pw1to),
                     (pw1me, pw1mo)], pb1_r, pw2_r, pb2_r)
            msk = (row >= b1) & (row < b2)
            y_ref[...] = jnp.where(msk, y, y_ref[...])

        @pl.when((start < b3) & (end > b2))
        def _():
            y = mlp([a_ref[...], m_ref[...]],
                    [(tw1ae, tw1ao), (tw1me, tw1mo)], tb1_r, tw2_r, tb2_r)
            msk = (row >= b2) & (row < b3)
            y_ref[...] = jnp.where(msk, y, y_ref[...])

        @pl.when(end > b3)
        def _():
            y = mlp([a_ref[...], m_ref[...]],
                    [(mw1ae, mw1ao), (mw1me, mw1mo)], mb1_r, mw2_r, mb2_r)
            y_ref[...] = jnp.where(row >= b3, y, y_ref[...])

    xspec = pl.BlockSpec((BLK, CW), lambda g: (g, 0))
    wfull = pl.BlockSpec((CW, H), lambda g: (0, 0))
    w2full = pl.BlockSpec((H, OUT), lambda g: (0, 0))
    bvec = pl.BlockSpec((1, H), lambda g: (0, 0))
    bvec2 = pl.BlockSpec((1, OUT), lambda g: (0, 0))

    return pl.pallas_call(
        body,
        grid=(NBLK,),
        in_specs=[
            pl.BlockSpec(memory_space=pltpu.SMEM),
            xspec, xspec, xspec, xspec,
            bvec2,
            wfull, wfull, wfull, wfull, wfull, wfull, wfull, wfull,
            bvec, w2full, bvec2,
            wfull, wfull, wfull, wfull, bvec, w2full, bvec2,
            wfull, wfull, wfull, wfull, bvec, w2full, bvec2,
        ],
        out_specs=pl.BlockSpec((BLK, OUT), lambda g: (g, 0)),
        out_shape=jax.ShapeDtypeStruct((N, OUT), jnp.float32),
    )(bounds, a, f, t, m, wait_row,
      *pw1, pb1, pw2, pb2,
      *tw1, tb1, tw2, tb2,
      *mw1, mb1, mw2, mb2)


def kernel(action_type, agv_idx, op_from_idx, op_to_idx, machine_idx, cu_seqlens,
           emb_AGV, emb_operation, emb_machine, wait_emb,
           pick_W1, pick_b1, pick_W2, pick_b2,
           trans_W1, trans_b1, trans_W2, trans_b2,
           move_W1, move_b1, move_W2, move_b2):
    at = action_type.astype(jnp.int32)
    order = jnp.argsort(at).astype(jnp.int32)
    inv = jnp.zeros((N,), jnp.int32).at[order].set(jnp.arange(N, dtype=jnp.int32))
    sorted_t = jnp.take(at, order)
    bounds = jnp.searchsorted(sorted_t, jnp.arange(1, 4, dtype=jnp.int32)).astype(jnp.int32)

    ia = jnp.take(agv_idx.astype(jnp.int32), order)
    if_ = jnp.take(op_from_idx.astype(jnp.int32), order)
    it = jnp.take(op_to_idx.astype(jnp.int32), order)
    im = jnp.take(machine_idx.astype(jnp.int32), order)

    a, f, t, m = _sc_gather4(ia, if_, it, im, _pack_bf16_words(emb_AGV),
                             _pack_bf16_words(emb_operation),
                             _pack_bf16_words(emb_machine))

    def _eo(w1, n_panels):
        out = []
        for i in range(n_panels):
            p = w1[i * C:(i + 1) * C]
            out.extend([p[0::2], p[1::2]])
        return out

    pw1 = _eo(pick_W1, 4)
    tw1 = _eo(trans_W1, 2)
    mw1 = _eo(move_W1, 2)

    y_sorted = _tc_mlps(
        a, f, t, m, bounds, wait_emb.reshape(1, OUT),
        pw1, pick_b1.reshape(1, H), pick_W2, pick_b2.reshape(1, OUT),
        tw1, trans_b1.reshape(1, H), trans_W2, trans_b2.reshape(1, OUT),
        mw1, move_b1.reshape(1, H), move_W2, move_b2.reshape(1, OUT))

    return _sc_permute_rows(y_sorted, inv)


# speedup vs baseline: 5.8164x; 5.8164x over previous
import functools

import jax
import jax.numpy as jnp
from jax import lax
from jax.experimental import pallas as pl
from jax.experimental.pallas import tpu as pltpu
from jax.experimental.pallas import tpu_sc as plsc

N = 16384
C = 256
H = 1024
OUT = 512
BLK = 1024
NBLK = N // BLK
NW = 32
ROWS_W = N // NW
CH = 64
NCH = ROWS_W // CH


def _sc_mesh():
    return plsc.VectorSubcoreMesh(core_axis_name="c", subcore_axis_name="s")


def _sc_gather4(ia, if_, it, im, tab_a, tab_o, tab_m, bounds_rep):
    @functools.partial(
        pl.kernel,
        mesh=_sc_mesh(),
        out_type=[jax.ShapeDtypeStruct((N, C), jnp.float32) for _ in range(4)],
        scratch_types=(
            [pltpu.VMEM((CH,), jnp.int32) for _ in range(4)]
            + [pltpu.VMEM((CH, C), jnp.float32) for _ in range(4)]
            + [pltpu.SemaphoreType.DMA for _ in range(4)]
            + [pltpu.VMEM((32,), jnp.int32)]
        ),
    )
    def k(ia_h, if_h, it_h, im_h, ta_h, to_h, tm_h, bnd_h,
          oa_h, of_h, ot_h, om_h,
          iv0, iv1, iv2, iv3, rv0, rv1, rv2, rv3, s0, s1, s2, s3, bv):
        wid = lax.axis_index("s") * 2 + lax.axis_index("c")
        base = wid * ROWS_W
        pltpu.sync_copy(bnd_h, bv)
        b1 = bv[pl.ds(0, 16)][0]
        b2 = bv[pl.ds(16, 16)][0]

        def chunk(c, carry):
            off = base + c * CH

            @pl.when(off + CH > b1)
            def _():
                pltpu.sync_copy(ia_h.at[pl.ds(off, CH)], iv0)
                pltpu.sync_copy(im_h.at[pl.ds(off, CH)], iv3)
                c0 = pltpu.async_copy(ta_h.at[iv0], rv0, s0)
                c3 = pltpu.async_copy(tm_h.at[iv3], rv3, s3)
                c0.wait()
                c3.wait()
                pltpu.sync_copy(rv0, oa_h.at[pl.ds(off, CH)])
                pltpu.sync_copy(rv3, om_h.at[pl.ds(off, CH)])

            @pl.when((off + CH > b1) & (off < b2))
            def _():
                pltpu.sync_copy(if_h.at[pl.ds(off, CH)], iv1)
                pltpu.sync_copy(it_h.at[pl.ds(off, CH)], iv2)
                c1 = pltpu.async_copy(to_h.at[iv1], rv1, s1)
                c2 = pltpu.async_copy(to_h.at[iv2], rv2, s2)
                c1.wait()
                c2.wait()
                pltpu.sync_copy(rv1, of_h.at[pl.ds(off, CH)])
                pltpu.sync_copy(rv2, ot_h.at[pl.ds(off, CH)])

            return carry

        lax.fori_loop(0, NCH, chunk, 0)

    return k(ia, if_, it, im, tab_a, tab_o, tab_m, bounds_rep)


def _sc_permute_rows(y, idx):
    @functools.partial(
        pl.kernel,
        mesh=_sc_mesh(),
        out_type=jax.ShapeDtypeStruct((N, OUT), jnp.float32),
        scratch_types=[
            pltpu.VMEM((CH,), jnp.int32),
            pltpu.VMEM((CH, OUT), jnp.float32),
            pltpu.SemaphoreType.DMA,
        ],
    )
    def k(y_h, idx_h, out_h, iv, rv, sem):
        wid = lax.axis_index("s") * 2 + lax.axis_index("c")
        base = wid * ROWS_W

        def chunk(c, carry):
            off = base + c * CH
            pltpu.sync_copy(idx_h.at[pl.ds(off, CH)], iv)
            pltpu.async_copy(y_h.at[iv], rv, sem).wait()
            pltpu.sync_copy(rv, out_h.at[pl.ds(off, CH)])
            return carry

        lax.fori_loop(0, NCH, chunk, 0)

    return k(y, idx)


def _tc_mlps(a, f, t, m, bounds, wait_row,
             pw1, pb1, pw2, pb2, tw1, tb1, tw2, tb2, mw1, mb1, mw2, mb2):
    def body(bounds_ref, a_ref, f_ref, t_ref, m_ref, wait_ref,
             pw1a, pw1f, pw1t, pw1m, pb1_r, pw2_r, pb2_r,
             tw1a, tw1m, tb1_r, tw2_r, tb2_r,
             mw1a, mw1m, mb1_r, mw2_r, mb2_r,
             y_ref):
        g = pl.program_id(0)
        start = g * BLK
        end = start + BLK
        b1 = bounds_ref[0]
        b2 = bounds_ref[1]
        b3 = bounds_ref[2]
        row = start + lax.broadcasted_iota(jnp.int32, (BLK, 1), 0)

        y_ref[...] = jnp.broadcast_to(wait_ref[...], (BLK, OUT))

        def mlp(parts, w1s, b1v, w2, b2v):
            acc = None
            for x, w in zip(parts, w1s):
                p = jnp.dot(x.astype(jnp.bfloat16), w[...],
                            preferred_element_type=jnp.float32)
                acc = p if acc is None else acc + p
            hh = acc + b1v[...]
            hh = jnp.where(hh >= 0.0, hh, 0.01 * hh)
            return jnp.dot(hh.astype(jnp.bfloat16), w2[...],
                           preferred_element_type=jnp.float32) + b2v[...]

        @pl.when((start < b2) & (end > b1))
        def _():
            y = mlp([a_ref[...], f_ref[...], t_ref[...], m_ref[...]],
                    [pw1a, pw1f, pw1t, pw1m], pb1_r, pw2_r, pb2_r)
            msk = (row >= b1) & (row < b2)
            y_ref[...] = jnp.where(msk, y, y_ref[...])

        @pl.when((start < b3) & (end > b2))
        def _():
            y = mlp([a_ref[...], m_ref[...]], [tw1a, tw1m], tb1_r, tw2_r, tb2_r)
            msk = (row >= b2) & (row < b3)
            y_ref[...] = jnp.where(msk, y, y_ref[...])

        @pl.when(end > b3)
        def _():
            y = mlp([a_ref[...], m_ref[...]], [mw1a, mw1m], mb1_r, mw2_r, mb2_r)
            y_ref[...] = jnp.where(row >= b3, y, y_ref[...])

    xspec = pl.BlockSpec((BLK, C), lambda g: (g, 0))
    wfull = pl.BlockSpec((C, H), lambda g: (0, 0))
    w2full = pl.BlockSpec((H, OUT), lambda g: (0, 0))
    bvec = pl.BlockSpec((1, H), lambda g: (0, 0))
    bvec2 = pl.BlockSpec((1, OUT), lambda g: (0, 0))

    return pl.pallas_call(
        body,
        grid=(NBLK,),
        in_specs=[
            pl.BlockSpec(memory_space=pltpu.SMEM),
            xspec, xspec, xspec, xspec,
            bvec2,
            wfull, wfull, wfull, wfull, bvec, w2full, bvec2,
            wfull, wfull, bvec, w2full, bvec2,
            wfull, wfull, bvec, w2full, bvec2,
        ],
        out_specs=pl.BlockSpec((BLK, OUT), lambda g: (g, 0)),
        out_shape=jax.ShapeDtypeStruct((N, OUT), jnp.float32),
    )(bounds, a, f, t, m, wait_row,
      pw1[0], pw1[1], pw1[2], pw1[3], pb1, pw2, pb2,
      tw1[0], tw1[1], tb1, tw2, tb2,
      mw1[0], mw1[1], mb1, mw2, mb2)


def kernel(action_type, agv_idx, op_from_idx, op_to_idx, machine_idx, cu_seqlens,
           emb_AGV, emb_operation, emb_machine, wait_emb,
           pick_W1, pick_b1, pick_W2, pick_b2,
           trans_W1, trans_b1, trans_W2, trans_b2,
           move_W1, move_b1, move_W2, move_b2):
    at = action_type.astype(jnp.int32)
    order = jnp.argsort(at).astype(jnp.int32)
    inv = jnp.zeros((N,), jnp.int32).at[order].set(jnp.arange(N, dtype=jnp.int32))
    sorted_t = jnp.take(at, order)
    bounds = jnp.searchsorted(sorted_t, jnp.arange(1, 4, dtype=jnp.int32)).astype(jnp.int32)

    ia = jnp.take(agv_idx.astype(jnp.int32), order)
    if_ = jnp.take(op_from_idx.astype(jnp.int32), order)
    it = jnp.take(op_to_idx.astype(jnp.int32), order)
    im = jnp.take(machine_idx.astype(jnp.int32), order)

    bounds_rep = jnp.repeat(bounds[:2], 16).astype(jnp.int32)
    a, f, t, m = _sc_gather4(ia, if_, it, im, emb_AGV, emb_operation,
                             emb_machine, bounds_rep)

    bf = jnp.bfloat16
    pw1 = [pick_W1[i * C:(i + 1) * C].astype(bf) for i in range(4)]
    tw1 = [trans_W1[i * C:(i + 1) * C].astype(bf) for i in range(2)]
    mw1 = [move_W1[i * C:(i + 1) * C].astype(bf) for i in range(2)]

    y_sorted = _tc_mlps(
        a, f, t, m, bounds, wait_emb.reshape(1, OUT),
        pw1, pick_b1.reshape(1, H), pick_W2.astype(bf), pick_b2.reshape(1, OUT),
        tw1, trans_b1.reshape(1, H), trans_W2.astype(bf), trans_b2.reshape(1, OUT),
        mw1, move_b1.reshape(1, H), move_W2.astype(bf), move_b2.reshape(1, OUT))

    return _sc_permute_rows(y_sorted, inv)


# speedup vs baseline: 5.8373x; 1.0036x over previous
import functools

import jax
import jax.numpy as jnp
from jax import lax
from jax.experimental import pallas as pl
from jax.experimental.pallas import tpu as pltpu
from jax.experimental.pallas import tpu_sc as plsc

N = 16384
C = 256
H = 1024
OUT = 512
BLK = 1024
NBLK = N // BLK
NW = 32
ROWS_W = N // NW
CH = 64
NCH = ROWS_W // CH


def _sc_mesh():
    return plsc.VectorSubcoreMesh(core_axis_name="c", subcore_axis_name="s")


def _sc_gather4(ia, if_, it, im, tab_a, tab_o, tab_m, bounds_rep):
    @functools.partial(
        pl.kernel,
        mesh=_sc_mesh(),
        out_type=[jax.ShapeDtypeStruct((N, C), jnp.float32) for _ in range(4)],
        scratch_types=(
            [pltpu.VMEM((CH,), jnp.int32) for _ in range(4)]
            + [pltpu.VMEM((CH, C), jnp.float32) for _ in range(4)]
            + [pltpu.SemaphoreType.DMA for _ in range(4)]
            + [pltpu.VMEM((32,), jnp.int32)]
        ),
    )
    def k(ia_h, if_h, it_h, im_h, ta_h, to_h, tm_h, bnd_h,
          oa_h, of_h, ot_h, om_h,
          iv0, iv1, iv2, iv3, rv0, rv1, rv2, rv3, s0, s1, s2, s3, bv):
        wid = lax.axis_index("s") * 2 + lax.axis_index("c")
        pltpu.sync_copy(bnd_h, bv)
        b1 = bv[pl.ds(0, 16)][0]
        b2 = bv[pl.ds(16, 16)][0]

        def chunk(c, carry):
            off = (c * NW + wid) * CH

            @pl.when(off + CH > b1)
            def _():
                pltpu.sync_copy(ia_h.at[pl.ds(off, CH)], iv0)
                pltpu.sync_copy(im_h.at[pl.ds(off, CH)], iv3)
                c0 = pltpu.async_copy(ta_h.at[iv0], rv0, s0)
                c3 = pltpu.async_copy(tm_h.at[iv3], rv3, s3)
                c0.wait()
                c3.wait()
                pltpu.sync_copy(rv0, oa_h.at[pl.ds(off, CH)])
                pltpu.sync_copy(rv3, om_h.at[pl.ds(off, CH)])

            @pl.when((off + CH > b1) & (off < b2))
            def _():
                pltpu.sync_copy(if_h.at[pl.ds(off, CH)], iv1)
                pltpu.sync_copy(it_h.at[pl.ds(off, CH)], iv2)
                c1 = pltpu.async_copy(to_h.at[iv1], rv1, s1)
                c2 = pltpu.async_copy(to_h.at[iv2], rv2, s2)
                c1.wait()
                c2.wait()
                pltpu.sync_copy(rv1, of_h.at[pl.ds(off, CH)])
                pltpu.sync_copy(rv2, ot_h.at[pl.ds(off, CH)])

            return carry

        lax.fori_loop(0, NCH, chunk, 0)

    return k(ia, if_, it, im, tab_a, tab_o, tab_m, bounds_rep)


def _sc_permute_rows(y, idx):
    @functools.partial(
        pl.kernel,
        mesh=_sc_mesh(),
        out_type=jax.ShapeDtypeStruct((N, OUT), jnp.float32),
        scratch_types=[
            pltpu.VMEM((CH,), jnp.int32),
            pltpu.VMEM((CH, OUT), jnp.float32),
            pltpu.SemaphoreType.DMA,
        ],
    )
    def k(y_h, idx_h, out_h, iv, rv, sem):
        wid = lax.axis_index("s") * 2 + lax.axis_index("c")
        base = wid * ROWS_W

        def chunk(c, carry):
            off = base + c * CH
            pltpu.sync_copy(idx_h.at[pl.ds(off, CH)], iv)
            pltpu.async_copy(y_h.at[iv], rv, sem).wait()
            pltpu.sync_copy(rv, out_h.at[pl.ds(off, CH)])
            return carry

        lax.fori_loop(0, NCH, chunk, 0)

    return k(y, idx)


def _tc_mlps(a, f, t, m, bounds, wait_row,
             pw1, pb1, pw2, pb2, tw1, tb1, tw2, tb2, mw1, mb1, mw2, mb2):
    def body(bounds_ref, a_ref, f_ref, t_ref, m_ref, wait_ref,
             pw1a, pw1f, pw1t, pw1m, pb1_r, pw2_r, pb2_r,
             tw1a, tw1m, tb1_r, tw2_r, tb2_r,
             mw1a, mw1m, mb1_r, mw2_r, mb2_r,
             y_ref):
        g = pl.program_id(0)
        start = g * BLK
        end = start + BLK
        b1 = bounds_ref[0]
        b2 = bounds_ref[1]
        b3 = bounds_ref[2]
        row = start + lax.broadcasted_iota(jnp.int32, (BLK, 1), 0)

        y_ref[...] = jnp.broadcast_to(wait_ref[...], (BLK, OUT))

        def mlp(parts, w1s, b1v, w2, b2v):
            acc = None
            for x, w in zip(parts, w1s):
                p = jnp.dot(x.astype(jnp.bfloat16), w[...],
                            preferred_element_type=jnp.float32)
                acc = p if acc is None else acc + p
            hh = acc + b1v[...]
            hh = jnp.where(hh >= 0.0, hh, 0.01 * hh)
            return jnp.dot(hh.astype(jnp.bfloat16), w2[...],
                           preferred_element_type=jnp.float32) + b2v[...]

        @pl.when((start < b2) & (end > b1))
        def _():
            y = mlp([a_ref[...], f_ref[...], t_ref[...], m_ref[...]],
                    [pw1a, pw1f, pw1t, pw1m], pb1_r, pw2_r, pb2_r)
            msk = (row >= b1) & (row < b2)
            y_ref[...] = jnp.where(msk, y, y_ref[...])

        @pl.when((start < b3) & (end > b2))
        def _():
            y = mlp([a_ref[...], m_ref[...]], [tw1a, tw1m], tb1_r, tw2_r, tb2_r)
            msk = (row >= b2) & (row < b3)
            y_ref[...] = jnp.where(msk, y, y_ref[...])

        @pl.when(end > b3)
        def _():
            y = mlp([a_ref[...], m_ref[...]], [mw1a, mw1m], mb1_r, mw2_r, mb2_r)
            y_ref[...] = jnp.where(row >= b3, y, y_ref[...])

    xspec = pl.BlockSpec((BLK, C), lambda g: (g, 0))
    wfull = pl.BlockSpec((C, H), lambda g: (0, 0))
    w2full = pl.BlockSpec((H, OUT), lambda g: (0, 0))
    bvec = pl.BlockSpec((1, H), lambda g: (0, 0))
    bvec2 = pl.BlockSpec((1, OUT), lambda g: (0, 0))

    return pl.pallas_call(
        body,
        grid=(NBLK,),
        in_specs=[
            pl.BlockSpec(memory_space=pltpu.SMEM),
            xspec, xspec, xspec, xspec,
            bvec2,
            wfull, wfull, wfull, wfull, bvec, w2full, bvec2,
            wfull, wfull, bvec, w2full, bvec2,
            wfull, wfull, bvec, w2full, bvec2,
        ],
        out_specs=pl.BlockSpec((BLK, OUT), lambda g: (g, 0)),
        out_shape=jax.ShapeDtypeStruct((N, OUT), jnp.float32),
    )(bounds, a, f, t, m, wait_row,
      pw1[0], pw1[1], pw1[2], pw1[3], pb1, pw2, pb2,
      tw1[0], tw1[1], tb1, tw2, tb2,
      mw1[0], mw1[1], mb1, mw2, mb2)


def kernel(action_type, agv_idx, op_from_idx, op_to_idx, machine_idx, cu_seqlens,
           emb_AGV, emb_operation, emb_machine, wait_emb,
           pick_W1, pick_b1, pick_W2, pick_b2,
           trans_W1, trans_b1, trans_W2, trans_b2,
           move_W1, move_b1, move_W2, move_b2):
    at = action_type.astype(jnp.int32)
    order = jnp.argsort(at).astype(jnp.int32)
    inv = jnp.zeros((N,), jnp.int32).at[order].set(jnp.arange(N, dtype=jnp.int32))
    sorted_t = jnp.take(at, order)
    bounds = jnp.searchsorted(sorted_t, jnp.arange(1, 4, dtype=jnp.int32)).astype(jnp.int32)

    ia = jnp.take(agv_idx.astype(jnp.int32), order)
    if_ = jnp.take(op_from_idx.astype(jnp.int32), order)
    it = jnp.take(op_to_idx.astype(jnp.int32), order)
    im = jnp.take(machine_idx.astype(jnp.int32), order)

    bounds_rep = jnp.repeat(bounds[:2], 16).astype(jnp.int32)
    a, f, t, m = _sc_gather4(ia, if_, it, im, emb_AGV, emb_operation,
                             emb_machine, bounds_rep)

    bf = jnp.bfloat16
    pw1 = [pick_W1[i * C:(i + 1) * C].astype(bf) for i in range(4)]
    tw1 = [trans_W1[i * C:(i + 1) * C].astype(bf) for i in range(2)]
    mw1 = [move_W1[i * C:(i + 1) * C].astype(bf) for i in range(2)]

    y_sorted = _tc_mlps(
        a, f, t, m, bounds, wait_emb.reshape(1, OUT),
        pw1, pick_b1.reshape(1, H), pick_W2.astype(bf), pick_b2.reshape(1, OUT),
        tw1, trans_b1.reshape(1, H), trans_W2.astype(bf), trans_b2.reshape(1, OUT),
        mw1, move_b1.reshape(1, H), move_W2.astype(bf), move_b2.reshape(1, OUT))

    return _sc_permute_rows(y_sorted, inv)


# speedup vs baseline: 5.9142x; 1.0132x over previous
import functools

import jax
import jax.numpy as jnp
from jax import lax
from jax.experimental import pallas as pl
from jax.experimental.pallas import tpu as pltpu
from jax.experimental.pallas import tpu_sc as plsc

N = 16384
C = 256
H = 1024
OUT = 512
BLK = 1024
NBLK = N // BLK
NW = 32
ROWS_W = N // NW
CH = 64
NCH = ROWS_W // CH


def _sc_mesh():
    return plsc.VectorSubcoreMesh(core_axis_name="c", subcore_axis_name="s")


def _sc_gather4(ia, if_, it, im, tab_a, tab_o, tab_m, bounds_rep):
    @functools.partial(
        pl.kernel,
        mesh=_sc_mesh(),
        out_type=[jax.ShapeDtypeStruct((N, C), jnp.float32) for _ in range(4)],
        scratch_types=(
            [pltpu.VMEM((CH,), jnp.int32) for _ in range(4)]
            + [pltpu.VMEM((CH, C), jnp.float32) for _ in range(4)]
            + [pltpu.SemaphoreType.DMA for _ in range(4)]
            + [pltpu.VMEM((32,), jnp.int32)]
        ),
    )
    def k(ia_h, if_h, it_h, im_h, ta_h, to_h, tm_h, bnd_h,
          oa_h, of_h, ot_h, om_h,
          iv0, iv1, iv2, iv3, rv0, rv1, rv2, rv3, s0, s1, s2, s3, bv):
        wid = lax.axis_index("s") * 2 + lax.axis_index("c")
        pltpu.sync_copy(bnd_h, bv)
        b1 = bv[pl.ds(0, 16)][0]
        b2 = bv[pl.ds(16, 16)][0]

        def chunk(c, carry):
            off = (c * NW + wid) * CH

            @pl.when(off + CH > b1)
            def _():
                pltpu.sync_copy(ia_h.at[pl.ds(off, CH)], iv0)
                pltpu.sync_copy(im_h.at[pl.ds(off, CH)], iv3)
                c0 = pltpu.async_copy(ta_h.at[iv0], rv0, s0)
                c3 = pltpu.async_copy(tm_h.at[iv3], rv3, s3)
                c0.wait()
                c3.wait()
                pltpu.sync_copy(rv0, oa_h.at[pl.ds(off, CH)])
                pltpu.sync_copy(rv3, om_h.at[pl.ds(off, CH)])

            @pl.when((off + CH > b1) & (off < b2))
            def _():
                pltpu.sync_copy(if_h.at[pl.ds(off, CH)], iv1)
                pltpu.sync_copy(it_h.at[pl.ds(off, CH)], iv2)
                c1 = pltpu.async_copy(to_h.at[iv1], rv1, s1)
                c2 = pltpu.async_copy(to_h.at[iv2], rv2, s2)
                c1.wait()
                c2.wait()
                pltpu.sync_copy(rv1, of_h.at[pl.ds(off, CH)])
                pltpu.sync_copy(rv2, ot_h.at[pl.ds(off, CH)])

            return carry

        lax.fori_loop(0, NCH, chunk, 0)

    return k(ia, if_, it, im, tab_a, tab_o, tab_m, bounds_rep)


def _sc_permute_rows(y, idx):
    @functools.partial(
        pl.kernel,
        mesh=_sc_mesh(),
        out_type=jax.ShapeDtypeStruct((N, OUT), jnp.float32),
        scratch_types=[
            pltpu.VMEM((CH,), jnp.int32),
            pltpu.VMEM((CH, OUT), jnp.float32),
            pltpu.SemaphoreType.DMA,
        ],
    )
    def k(y_h, idx_h, out_h, iv, rv, sem):
        wid = lax.axis_index("s") * 2 + lax.axis_index("c")
        base = wid * ROWS_W

        def chunk(c, carry):
            off = base + c * CH
            pltpu.sync_copy(idx_h.at[pl.ds(off, CH)], iv)
            pltpu.async_copy(y_h.at[iv], rv, sem).wait()
            pltpu.sync_copy(rv, out_h.at[pl.ds(off, CH)])
            return carry

        lax.fori_loop(0, NCH, chunk, 0)

    return k(y, idx)


def _tc_mlps(a, f, t, m, bounds, wait_row,
             pw1, pb1, pw2, pb2, tw1, tb1, tw2, tb2, mw1, mb1, mw2, mb2):
    def body(bounds_ref, a_ref, f_ref, t_ref, m_ref, wait_ref,
             pw1a, pw1f, pw1t, pw1m, pb1_r, pw2_r, pb2_r,
             tw1a, tw1m, tb1_r, tw2_r, tb2_r,
             mw1a, mw1m, mb1_r, mw2_r, mb2_r,
             y_ref):
        g = pl.program_id(0)
        start = g * BLK
        end = start + BLK
        b1 = bounds_ref[0]
        b2 = bounds_ref[1]
        b3 = bounds_ref[2]
        row = start + lax.broadcasted_iota(jnp.int32, (BLK, 1), 0)

        y_ref[...] = jnp.broadcast_to(wait_ref[...], (BLK, OUT))

        def mlp(parts, w1s, b1v, w2, b2v):
            acc = None
            for x, w in zip(parts, w1s):
                p = jnp.dot(x.astype(jnp.bfloat16), w[...],
                            preferred_element_type=jnp.float32)
                acc = p if acc is None else acc + p
            hh = acc + b1v[...]
            hh = jnp.where(hh >= 0.0, hh, 0.01 * hh)
            return jnp.dot(hh.astype(jnp.bfloat16), w2[...],
                           preferred_element_type=jnp.float32) + b2v[...]

        @pl.when((start < b2) & (end > b1))
        def _():
            y = mlp([a_ref[...], f_ref[...], t_ref[...], m_ref[...]],
                    [pw1a, pw1f, pw1t, pw1m], pb1_r, pw2_r, pb2_r)
            msk = (row >= b1) & (row < b2)
            y_ref[...] = jnp.where(msk, y, y_ref[...])

        @pl.when((start < b3) & (end > b2))
        def _():
            y = mlp([a_ref[...], m_ref[...]], [tw1a, tw1m], tb1_r, tw2_r, tb2_r)
            msk = (row >= b2) & (row < b3)
            y_ref[...] = jnp.where(msk, y, y_ref[...])

        @pl.when(end > b3)
        def _():
            y = mlp([a_ref[...], m_ref[...]], [mw1a, mw1m], mb1_r, mw2_r, mb2_r)
            y_ref[...] = jnp.where(row >= b3, y, y_ref[...])

    def am_map(g, b):
        return jnp.maximum(g, b[0] // BLK), 0

    def ft_map(g, b):
        lo = b[0] // BLK
        hi = jnp.maximum(lo, (b[1] - 1) // BLK)
        return jnp.clip(g, lo, hi), 0

    xspec_am = pl.BlockSpec((BLK, C), am_map)
    xspec_ft = pl.BlockSpec((BLK, C), ft_map)
    wfull = pl.BlockSpec((C, H), lambda g, b: (0, 0))
    w2full = pl.BlockSpec((H, OUT), lambda g, b: (0, 0))
    bvec = pl.BlockSpec((1, H), lambda g, b: (0, 0))
    bvec2 = pl.BlockSpec((1, OUT), lambda g, b: (0, 0))

    grid_spec = pltpu.PrefetchScalarGridSpec(
        num_scalar_prefetch=1,
        grid=(NBLK,),
        in_specs=[
            xspec_am, xspec_ft, xspec_ft, xspec_am,
            bvec2,
            wfull, wfull, wfull, wfull, bvec, w2full, bvec2,
            wfull, wfull, bvec, w2full, bvec2,
            wfull, wfull, bvec, w2full, bvec2,
        ],
        out_specs=pl.BlockSpec((BLK, OUT), lambda g, b: (g, 0)),
    )
    return pl.pallas_call(
        body,
        grid_spec=grid_spec,
        out_shape=jax.ShapeDtypeStruct((N, OUT), jnp.float32),
    )(bounds, a, f, t, m, wait_row,
      pw1[0], pw1[1], pw1[2], pw1[3], pb1, pw2, pb2,
      tw1[0], tw1[1], tb1, tw2, tb2,
      mw1[0], mw1[1], mb1, mw2, mb2)


def kernel(action_type, agv_idx, op_from_idx, op_to_idx, machine_idx, cu_seqlens,
           emb_AGV, emb_operation, emb_machine, wait_emb,
           pick_W1, pick_b1, pick_W2, pick_b2,
           trans_W1, trans_b1, trans_W2, trans_b2,
           move_W1, move_b1, move_W2, move_b2):
    at = action_type.astype(jnp.int32)
    order = jnp.argsort(at).astype(jnp.int32)
    inv = jnp.zeros((N,), jnp.int32).at[order].set(jnp.arange(N, dtype=jnp.int32))
    sorted_t = jnp.take(at, order)
    bounds = jnp.searchsorted(sorted_t, jnp.arange(1, 4, dtype=jnp.int32)).astype(jnp.int32)

    ia = jnp.take(agv_idx.astype(jnp.int32), order)
    if_ = jnp.take(op_from_idx.astype(jnp.int32), order)
    it = jnp.take(op_to_idx.astype(jnp.int32), order)
    im = jnp.take(machine_idx.astype(jnp.int32), order)

    bounds_rep = jnp.repeat(bounds[:2], 16).astype(jnp.int32)
    a, f, t, m = _sc_gather4(ia, if_, it, im, emb_AGV, emb_operation,
                             emb_machine, bounds_rep)

    bf = jnp.bfloat16
    pw1 = [pick_W1[i * C:(i + 1) * C].astype(bf) for i in range(4)]
    tw1 = [trans_W1[i * C:(i + 1) * C].astype(bf) for i in range(2)]
    mw1 = [move_W1[i * C:(i + 1) * C].astype(bf) for i in range(2)]

    y_sorted = _tc_mlps(
        a, f, t, m, bounds, wait_emb.reshape(1, OUT),
        pw1, pick_b1.reshape(1, H), pick_W2.astype(bf), pick_b2.reshape(1, OUT),
        tw1, trans_b1.reshape(1, H), trans_W2.astype(bf), trans_b2.reshape(1, OUT),
        mw1, move_b1.reshape(1, H), move_W2.astype(bf), move_b2.reshape(1, OUT))

    return _sc_permute_rows(y_sorted, inv)


# speedup vs baseline: 6.7203x; 1.1363x over previous
import functools

import jax
import jax.numpy as jnp
from jax import lax
from jax.experimental import pallas as pl
from jax.experimental.pallas import tpu as pltpu
from jax.experimental.pallas import tpu_sc as plsc

N = 16384
C = 256
H = 1024
OUT = 512
BLK = 1024
NBLK = N // BLK
NW = 32
ROWS_W = N // NW
CH = 64
NCH = ROWS_W // CH


def _sc_mesh():
    return plsc.VectorSubcoreMesh(core_axis_name="c", subcore_axis_name="s")


def _sc_gather4(order, ia, if_, it, im, tab_a, tab_o, tab_m, bounds_rep):
    @functools.partial(
        pl.kernel,
        mesh=_sc_mesh(),
        out_type=[jax.ShapeDtypeStruct((N, C), jnp.float32) for _ in range(4)],
        scratch_types=(
            [pltpu.VMEM((CH,), jnp.int32) for _ in range(5)]
            + [pltpu.VMEM((CH, C), jnp.float32) for _ in range(4)]
            + [pltpu.SemaphoreType.DMA for _ in range(5)]
            + [pltpu.VMEM((32,), jnp.int32)]
        ),
    )
    def k(ord_h, ia_h, if_h, it_h, im_h, ta_h, to_h, tm_h, bnd_h,
          oa_h, of_h, ot_h, om_h,
          ov, iv0, iv1, iv2, iv3, rv0, rv1, rv2, rv3,
          so, s0, s1, s2, s3, bv):
        wid = lax.axis_index("s") * 2 + lax.axis_index("c")
        pltpu.sync_copy(bnd_h, bv)
        b1 = bv[pl.ds(0, 16)][0]
        b2 = bv[pl.ds(16, 16)][0]

        def chunk(c, carry):
            off = (c * NW + wid) * CH

            @pl.when(off + CH > b1)
            def _():
                pltpu.sync_copy(ord_h.at[pl.ds(off, CH)], ov)
                g0 = pltpu.async_copy(ia_h.at[ov], iv0, so)
                g3 = pltpu.async_copy(im_h.at[ov], iv3, s0)
                g0.wait()
                g3.wait()
                c0 = pltpu.async_copy(ta_h.at[iv0], rv0, s0)
                c3 = pltpu.async_copy(tm_h.at[iv3], rv3, s3)

                @pl.when(off < b2)
                def _():
                    g1 = pltpu.async_copy(if_h.at[ov], iv1, s1)
                    g2 = pltpu.async_copy(it_h.at[ov], iv2, s2)
                    g1.wait()
                    g2.wait()
                    c1 = pltpu.async_copy(to_h.at[iv1], rv1, s1)
                    c2 = pltpu.async_copy(to_h.at[iv2], rv2, s2)
                    c1.wait()
                    c2.wait()
                    pltpu.sync_copy(rv1, of_h.at[pl.ds(off, CH)])
                    pltpu.sync_copy(rv2, ot_h.at[pl.ds(off, CH)])

                c0.wait()
                c3.wait()
                pltpu.sync_copy(rv0, oa_h.at[pl.ds(off, CH)])
                pltpu.sync_copy(rv3, om_h.at[pl.ds(off, CH)])

            return carry

        lax.fori_loop(0, NCH, chunk, 0)

    return k(order, ia, if_, it, im, tab_a, tab_o, tab_m, bounds_rep)


def _sc_permute_rows(y, idx):
    @functools.partial(
        pl.kernel,
        mesh=_sc_mesh(),
        out_type=jax.ShapeDtypeStruct((N, OUT), jnp.float32),
        scratch_types=[
            pltpu.VMEM((CH,), jnp.int32),
            pltpu.VMEM((CH, OUT), jnp.float32),
            pltpu.SemaphoreType.DMA,
        ],
    )
    def k(y_h, idx_h, out_h, iv, rv, sem):
        wid = lax.axis_index("s") * 2 + lax.axis_index("c")
        base = wid * ROWS_W

        def chunk(c, carry):
            off = base + c * CH
            pltpu.sync_copy(idx_h.at[pl.ds(off, CH)], iv)
            pltpu.async_copy(y_h.at[iv], rv, sem).wait()
            pltpu.sync_copy(rv, out_h.at[pl.ds(off, CH)])
            return carry

        lax.fori_loop(0, NCH, chunk, 0)

    return k(y, idx)


def _tc_mlps(a, f, t, m, bounds, wait_row,
             pw1, pb1, pw2, pb2, tw1, tb1, tw2, tb2, mw1, mb1, mw2, mb2):
    def body(bounds_ref, a_ref, f_ref, t_ref, m_ref, wait_ref,
             pw1a, pw1f, pw1t, pw1m, pb1_r, pw2_r, pb2_r,
             tw1a, tw1m, tb1_r, tw2_r, tb2_r,
             mw1a, mw1m, mb1_r, mw2_r, mb2_r,
             y_ref):
        g = pl.program_id(0)
        start = g * BLK
        end = start + BLK
        b1 = bounds_ref[0]
        b2 = bounds_ref[1]
        b3 = bounds_ref[2]
        row = start + lax.broadcasted_iota(jnp.int32, (BLK, 1), 0)

        y_ref[...] = jnp.broadcast_to(wait_ref[...], (BLK, OUT))

        def mlp(parts, w1s, b1v, w2, b2v):
            acc = None
            for x, w in zip(parts, w1s):
                p = jnp.dot(x.astype(jnp.bfloat16), w[...],
                            preferred_element_type=jnp.float32)
                acc = p if acc is None else acc + p
            hh = acc + b1v[...]
            hh = jnp.where(hh >= 0.0, hh, 0.01 * hh)
            return jnp.dot(hh.astype(jnp.bfloat16), w2[...],
                           preferred_element_type=jnp.float32) + b2v[...]

        @pl.when((start < b2) & (end > b1))
        def _():
            y = mlp([a_ref[...], f_ref[...], t_ref[...], m_ref[...]],
                    [pw1a, pw1f, pw1t, pw1m], pb1_r, pw2_r, pb2_r)
            msk = (row >= b1) & (row < b2)
            y_ref[...] = jnp.where(msk, y, y_ref[...])

        @pl.when((start < b3) & (end > b2))
        def _():
            y = mlp([a_ref[...], m_ref[...]], [tw1a, tw1m], tb1_r, tw2_r, tb2_r)
            msk = (row >= b2) & (row < b3)
            y_ref[...] = jnp.where(msk, y, y_ref[...])

        @pl.when(end > b3)
        def _():
            y = mlp([a_ref[...], m_ref[...]], [mw1a, mw1m], mb1_r, mw2_r, mb2_r)
            y_ref[...] = jnp.where(row >= b3, y, y_ref[...])

    def am_map(g, b):
        return jnp.maximum(g, b[0] // BLK), 0

    def ft_map(g, b):
        lo = b[0] // BLK
        hi = jnp.maximum(lo, (b[1] - 1) // BLK)
        return jnp.clip(g, lo, hi), 0

    xspec_am = pl.BlockSpec((BLK, C), am_map)
    xspec_ft = pl.BlockSpec((BLK, C), ft_map)
    wfull = pl.BlockSpec((C, H), lambda g, b: (0, 0))
    w2full = pl.BlockSpec((H, OUT), lambda g, b: (0, 0))
    bvec = pl.BlockSpec((1, H), lambda g, b: (0, 0))
    bvec2 = pl.BlockSpec((1, OUT), lambda g, b: (0, 0))

    grid_spec = pltpu.PrefetchScalarGridSpec(
        num_scalar_prefetch=1,
        grid=(NBLK,),
        in_specs=[
            xspec_am, xspec_ft, xspec_ft, xspec_am,
            bvec2,
            wfull, wfull, wfull, wfull, bvec, w2full, bvec2,
            wfull, wfull, bvec, w2full, bvec2,
            wfull, wfull, bvec, w2full, bvec2,
        ],
        out_specs=pl.BlockSpec((BLK, OUT), lambda g, b: (g, 0)),
    )
    return pl.pallas_call(
        body,
        grid_spec=grid_spec,
        out_shape=jax.ShapeDtypeStruct((N, OUT), jnp.float32),
    )(bounds, a, f, t, m, wait_row,
      pw1[0], pw1[1], pw1[2], pw1[3], pb1, pw2, pb2,
      tw1[0], tw1[1], tb1, tw2, tb2,
      mw1[0], mw1[1], mb1, mw2, mb2)


def kernel(action_type, agv_idx, op_from_idx, op_to_idx, machine_idx, cu_seqlens,
           emb_AGV, emb_operation, emb_machine, wait_emb,
           pick_W1, pick_b1, pick_W2, pick_b2,
           trans_W1, trans_b1, trans_W2, trans_b2,
           move_W1, move_b1, move_W2, move_b2):
    at = action_type.astype(jnp.int32)
    order = jnp.argsort(at).astype(jnp.int32)
    inv = jnp.zeros((N,), jnp.int32).at[order].set(jnp.arange(N, dtype=jnp.int32))
    sorted_t = jnp.take(at, order)
    bounds = jnp.searchsorted(sorted_t, jnp.arange(1, 4, dtype=jnp.int32)).astype(jnp.int32)

    bounds_rep = jnp.repeat(bounds[:2], 16).astype(jnp.int32)
    a, f, t, m = _sc_gather4(order, agv_idx.astype(jnp.int32),
                             op_from_idx.astype(jnp.int32),
                             op_to_idx.astype(jnp.int32),
                             machine_idx.astype(jnp.int32),
                             emb_AGV, emb_operation, emb_machine, bounds_rep)

    bf = jnp.bfloat16
    pw1 = [pick_W1[i * C:(i + 1) * C].astype(bf) for i in range(4)]
    tw1 = [trans_W1[i * C:(i + 1) * C].astype(bf) for i in range(2)]
    mw1 = [move_W1[i * C:(i + 1) * C].astype(bf) for i in range(2)]

    y_sorted = _tc_mlps(
        a, f, t, m, bounds, wait_emb.reshape(1, OUT),
        pw1, pick_b1.reshape(1, H), pick_W2.astype(bf), pick_b2.reshape(1, OUT),
        tw1, trans_b1.reshape(1, H), trans_W2.astype(bf), trans_b2.reshape(1, OUT),
        mw1, move_b1.reshape(1, H), move_W2.astype(bf), move_b2.reshape(1, OUT))

    return _sc_permute_rows(y_sorted, inv)


# speedup vs baseline: 7.8105x; 1.1622x over previous
import functools

import jax
import jax.numpy as jnp
from jax import lax
from jax.experimental import pallas as pl
from jax.experimental.pallas import tpu as pltpu
from jax.experimental.pallas import tpu_sc as plsc

N = 16384
C = 256
H = 1024
OUT = 512
BLK = 1024
NBLK = N // BLK
NW = 32
ROWS_W = N // NW
CH = 64
NCH = ROWS_W // CH


def _sc_mesh():
    return plsc.VectorSubcoreMesh(core_axis_name="c", subcore_axis_name="s")


def _sc_gather4(order, ia, if_, it, im, tab_a, tab_o, tab_m, bounds_rep):
    @functools.partial(
        pl.kernel,
        mesh=_sc_mesh(),
        out_type=[jax.ShapeDtypeStruct((N, C), jnp.float32) for _ in range(4)],
        scratch_types=(
            [pltpu.VMEM((CH,), jnp.int32) for _ in range(5)]
            + [pltpu.VMEM((CH, C), jnp.float32) for _ in range(4)]
            + [pltpu.SemaphoreType.DMA for _ in range(5)]
            + [pltpu.VMEM((32,), jnp.int32)]
        ),
    )
    def k(ord_h, ia_h, if_h, it_h, im_h, ta_h, to_h, tm_h, bnd_h,
          oa_h, of_h, ot_h, om_h,
          ov, iv0, iv1, iv2, iv3, rv0, rv1, rv2, rv3,
          so, s0, s1, s2, s3, bv):
        wid = lax.axis_index("s") * 2 + lax.axis_index("c")
        pltpu.sync_copy(bnd_h, bv)
        b1 = bv[pl.ds(0, 16)][0]
        b2 = bv[pl.ds(16, 16)][0]

        def chunk(c, carry):
            off = (c * NW + wid) * CH

            @pl.when(off + CH > b1)
            def _():
                pltpu.sync_copy(ord_h.at[pl.ds(off, CH)], ov)
                g0 = pltpu.async_copy(ia_h.at[ov], iv0, so)
                g3 = pltpu.async_copy(im_h.at[ov], iv3, s0)
                g0.wait()
                g3.wait()
                c0 = pltpu.async_copy(ta_h.at[iv0], rv0, s0)
                c3 = pltpu.async_copy(tm_h.at[iv3], rv3, s3)

                @pl.when(off < b2)
                def _():
                    g1 = pltpu.async_copy(if_h.at[ov], iv1, s1)
                    g2 = pltpu.async_copy(it_h.at[ov], iv2, s2)
                    g1.wait()
                    g2.wait()
                    c1 = pltpu.async_copy(to_h.at[iv1], rv1, s1)
                    c2 = pltpu.async_copy(to_h.at[iv2], rv2, s2)
                    c1.wait()
                    c2.wait()
                    pltpu.sync_copy(rv1, of_h.at[pl.ds(off, CH)])
                    pltpu.sync_copy(rv2, ot_h.at[pl.ds(off, CH)])

                c0.wait()
                c3.wait()
                pltpu.sync_copy(rv0, oa_h.at[pl.ds(off, CH)])
                pltpu.sync_copy(rv3, om_h.at[pl.ds(off, CH)])

            return carry

        lax.fori_loop(0, NCH, chunk, 0)

    return k(order, ia, if_, it, im, tab_a, tab_o, tab_m, bounds_rep)


def _sc_permute_rows(y, idx):
    @functools.partial(
        pl.kernel,
        mesh=_sc_mesh(),
        out_type=jax.ShapeDtypeStruct((N, OUT), jnp.float32),
        scratch_types=[
            pltpu.VMEM((CH,), jnp.int32),
            pltpu.VMEM((CH,), jnp.int32),
            pltpu.VMEM((CH, OUT), jnp.float32),
            pltpu.VMEM((CH, OUT), jnp.float32),
            pltpu.SemaphoreType.DMA,
            pltpu.SemaphoreType.DMA,
        ],
    )
    def k(y_h, idx_h, out_h, iv0, iv1, rv0, rv1, sm0, sm1):
        wid = lax.axis_index("s") * 2 + lax.axis_index("c")
        base = wid * ROWS_W

        def chunk(c, carry):
            off0 = base + (2 * c) * CH
            off1 = base + (2 * c + 1) * CH
            pltpu.sync_copy(idx_h.at[pl.ds(off0, CH)], iv0)
            pltpu.sync_copy(y_h.at[pl.ds(off0, CH)], rv0)
            sc0 = pltpu.async_copy(rv0, out_h.at[iv0], sm0)
            pltpu.sync_copy(idx_h.at[pl.ds(off1, CH)], iv1)
            pltpu.sync_copy(y_h.at[pl.ds(off1, CH)], rv1)
            sc1 = pltpu.async_copy(rv1, out_h.at[iv1], sm1)
            sc0.wait()
            sc1.wait()
            return carry

        lax.fori_loop(0, NCH // 2, chunk, 0)

    return k(y, idx)


def _tc_mlps(a, f, t, m, bounds, wait_row,
             pw1, pb1, pw2, pb2, tw1, tb1, tw2, tb2, mw1, mb1, mw2, mb2):
    def body(bounds_ref, a_ref, f_ref, t_ref, m_ref, wait_ref,
             pw1a, pw1f, pw1t, pw1m, pb1_r, pw2_r, pb2_r,
             tw1a, tw1m, tb1_r, tw2_r, tb2_r,
             mw1a, mw1m, mb1_r, mw2_r, mb2_r,
             y_ref):
        g = pl.program_id(0)
        start = g * BLK
        end = start + BLK
        b1 = bounds_ref[0]
        b2 = bounds_ref[1]
        b3 = bounds_ref[2]
        row = start + lax.broadcasted_iota(jnp.int32, (BLK, 1), 0)

        y_ref[...] = jnp.broadcast_to(wait_ref[...], (BLK, OUT))

        def mlp(parts, w1s, b1v, w2, b2v):
            acc = None
            for x, w in zip(parts, w1s):
                p = jnp.dot(x.astype(jnp.bfloat16), w[...],
                            preferred_element_type=jnp.float32)
                acc = p if acc is None else acc + p
            hh = acc + b1v[...]
            hh = jnp.where(hh >= 0.0, hh, 0.01 * hh)
            return jnp.dot(hh.astype(jnp.bfloat16), w2[...],
                           preferred_element_type=jnp.float32) + b2v[...]

        @pl.when((start < b2) & (end > b1))
        def _():
            y = mlp([a_ref[...], f_ref[...], t_ref[...], m_ref[...]],
                    [pw1a, pw1f, pw1t, pw1m], pb1_r, pw2_r, pb2_r)
            msk = (row >= b1) & (row < b2)
            y_ref[...] = jnp.where(msk, y, y_ref[...])

        @pl.when((start < b3) & (end > b2))
        def _():
            y = mlp([a_ref[...], m_ref[...]], [tw1a, tw1m], tb1_r, tw2_r, tb2_r)
            msk = (row >= b2) & (row < b3)
            y_ref[...] = jnp.where(msk, y, y_ref[...])

        @pl.when(end > b3)
        def _():
            y = mlp([a_ref[...], m_ref[...]], [mw1a, mw1m], mb1_r, mw2_r, mb2_r)
            y_ref[...] = jnp.where(row >= b3, y, y_ref[...])

    def am_map(g, b):
        return jnp.maximum(g, b[0] // BLK), 0

    def ft_map(g, b):
        lo = b[0] // BLK
        hi = jnp.maximum(lo, (b[1] - 1) // BLK)
        return jnp.clip(g, lo, hi), 0

    xspec_am = pl.BlockSpec((BLK, C), am_map)
    xspec_ft = pl.BlockSpec((BLK, C), ft_map)
    wfull = pl.BlockSpec((C, H), lambda g, b: (0, 0))
    w2full = pl.BlockSpec((H, OUT), lambda g, b: (0, 0))
    bvec = pl.BlockSpec((1, H), lambda g, b: (0, 0))
    bvec2 = pl.BlockSpec((1, OUT), lambda g, b: (0, 0))

    grid_spec = pltpu.PrefetchScalarGridSpec(
        num_scalar_prefetch=1,
        grid=(NBLK,),
        in_specs=[
            xspec_am, xspec_ft, xspec_ft, xspec_am,
            bvec2,
            wfull, wfull, wfull, wfull, bvec, w2full, bvec2,
            wfull, wfull, bvec, w2full, bvec2,
            wfull, wfull, bvec, w2full, bvec2,
        ],
        out_specs=pl.BlockSpec((BLK, OUT), lambda g, b: (g, 0)),
    )
    return pl.pallas_call(
        body,
        grid_spec=grid_spec,
        out_shape=jax.ShapeDtypeStruct((N, OUT), jnp.float32),
    )(bounds, a, f, t, m, wait_row,
      pw1[0], pw1[1], pw1[2], pw1[3], pb1, pw2, pb2,
      tw1[0], tw1[1], tb1, tw2, tb2,
      mw1[0], mw1[1], mb1, mw2, mb2)


def kernel(action_type, agv_idx, op_from_idx, op_to_idx, machine_idx, cu_seqlens,
           emb_AGV, emb_operation, emb_machine, wait_emb,
           pick_W1, pick_b1, pick_W2, pick_b2,
           trans_W1, trans_b1, trans_W2, trans_b2,
           move_W1, move_b1, move_W2, move_b2):
    at = action_type.astype(jnp.int32)
    order = jnp.argsort(at).astype(jnp.int32)
    sorted_t = jnp.take(at, order)
    bounds = jnp.searchsorted(sorted_t, jnp.arange(1, 4, dtype=jnp.int32)).astype(jnp.int32)

    bounds_rep = jnp.repeat(bounds[:2], 16).astype(jnp.int32)
    a, f, t, m = _sc_gather4(order, agv_idx.astype(jnp.int32),
                             op_from_idx.astype(jnp.int32),
                             op_to_idx.astype(jnp.int32),
                             machine_idx.astype(jnp.int32),
                             emb_AGV, emb_operation, emb_machine, bounds_rep)

    bf = jnp.bfloat16
    pw1 = [pick_W1[i * C:(i + 1) * C].astype(bf) for i in range(4)]
    tw1 = [trans_W1[i * C:(i + 1) * C].astype(bf) for i in range(2)]
    mw1 = [move_W1[i * C:(i + 1) * C].astype(bf) for i in range(2)]

    y_sorted = _tc_mlps(
        a, f, t, m, bounds, wait_emb.reshape(1, OUT),
        pw1, pick_b1.reshape(1, H), pick_W2.astype(bf), pick_b2.reshape(1, OUT),
        tw1, trans_b1.reshape(1, H), trans_W2.astype(bf), trans_b2.reshape(1, OUT),
        mw1, move_b1.reshape(1, H), move_W2.astype(bf), move_b2.reshape(1, OUT))

    return _sc_permute_rows(y_sorted, order)


# speedup vs baseline: 7.8249x; 1.0018x over previous
import functools

import jax
import jax.numpy as jnp
from jax import lax
from jax.experimental import pallas as pl
from jax.experimental.pallas import tpu as pltpu
from jax.experimental.pallas import tpu_sc as plsc

N = 16384
C = 256
H = 1024
OUT = 512
BLK = 1024
NBLK = N // BLK
NW = 32
ROWS_W = N // NW
CH = 64
NCH = ROWS_W // CH


def _sc_mesh():
    return plsc.VectorSubcoreMesh(core_axis_name="c", subcore_axis_name="s")


def _sc_gather4(order, ia, if_, it, im, tab_a, tab_o, tab_m, bounds_rep):
    @functools.partial(
        pl.kernel,
        mesh=_sc_mesh(),
        out_type=[jax.ShapeDtypeStruct((N, C), jnp.float32) for _ in range(4)],
        scratch_types=(
            [pltpu.VMEM((CH,), jnp.int32) for _ in range(5)]
            + [pltpu.VMEM((CH, C), jnp.float32) for _ in range(4)]
            + [pltpu.SemaphoreType.DMA for _ in range(5)]
            + [pltpu.VMEM((32,), jnp.int32)]
        ),
    )
    def k(ord_h, ia_h, if_h, it_h, im_h, ta_h, to_h, tm_h, bnd_h,
          oa_h, of_h, ot_h, om_h,
          ov, iv0, iv1, iv2, iv3, rv0, rv1, rv2, rv3,
          so, s0, s1, s2, s3, bv):
        wid = lax.axis_index("s") * 2 + lax.axis_index("c")
        pltpu.sync_copy(bnd_h, bv)
        b1 = bv[pl.ds(0, 16)][0]
        b2 = bv[pl.ds(16, 16)][0]

        def chunk(c, carry):
            off = (c * NW + wid) * CH

            @pl.when(off + CH > b1)
            def _():
                pltpu.sync_copy(ord_h.at[pl.ds(off, CH)], ov)
                g0 = pltpu.async_copy(ia_h.at[ov], iv0, so)
                g3 = pltpu.async_copy(im_h.at[ov], iv3, s0)
                g0.wait()
                g3.wait()
                c0 = pltpu.async_copy(ta_h.at[iv0], rv0, s0)
                c3 = pltpu.async_copy(tm_h.at[iv3], rv3, s3)

                @pl.when(off < b2)
                def _():
                    g1 = pltpu.async_copy(if_h.at[ov], iv1, s1)
                    g2 = pltpu.async_copy(it_h.at[ov], iv2, s2)
                    g1.wait()
                    g2.wait()
                    c1 = pltpu.async_copy(to_h.at[iv1], rv1, s1)
                    c2 = pltpu.async_copy(to_h.at[iv2], rv2, s2)
                    c1.wait()
                    c2.wait()
                    pltpu.sync_copy(rv1, of_h.at[pl.ds(off, CH)])
                    pltpu.sync_copy(rv2, ot_h.at[pl.ds(off, CH)])

                c0.wait()
                c3.wait()
                pltpu.sync_copy(rv0, oa_h.at[pl.ds(off, CH)])
                pltpu.sync_copy(rv3, om_h.at[pl.ds(off, CH)])

            return carry

        lax.fori_loop(0, NCH, chunk, 0)

    return k(order, ia, if_, it, im, tab_a, tab_o, tab_m, bounds_rep)


def _sc_permute_rows(y, idx):
    @functools.partial(
        pl.kernel,
        mesh=_sc_mesh(),
        out_type=jax.ShapeDtypeStruct((N, OUT), jnp.float32),
        scratch_types=[
            pltpu.VMEM((CH,), jnp.int32),
            pltpu.VMEM((CH,), jnp.int32),
            pltpu.VMEM((CH, OUT), jnp.float32),
            pltpu.VMEM((CH, OUT), jnp.float32),
            pltpu.SemaphoreType.DMA,
            pltpu.SemaphoreType.DMA,
        ],
    )
    def k(y_h, idx_h, out_h, iv0, iv1, rv0, rv1, sm0, sm1):
        wid = lax.axis_index("s") * 2 + lax.axis_index("c")
        base = wid * ROWS_W

        def chunk(c, carry):
            off0 = base + (2 * c) * CH
            off1 = base + (2 * c + 1) * CH
            pltpu.sync_copy(idx_h.at[pl.ds(off0, CH)], iv0)
            pltpu.sync_copy(y_h.at[pl.ds(off0, CH)], rv0)
            sc0 = pltpu.async_copy(rv0, out_h.at[iv0], sm0)
            pltpu.sync_copy(idx_h.at[pl.ds(off1, CH)], iv1)
            pltpu.sync_copy(y_h.at[pl.ds(off1, CH)], rv1)
            sc1 = pltpu.async_copy(rv1, out_h.at[iv1], sm1)
            sc0.wait()
            sc1.wait()
            return carry

        lax.fori_loop(0, NCH // 2, chunk, 0)

    return k(y, idx)


def _tc_mlps(a, f, t, m, bounds, wait_row,
             pw1, pb1, pw2, pb2, tw1, tb1, tw2, tb2, mw1, mb1, mw2, mb2):
    def body(bounds_ref, a_ref, f_ref, t_ref, m_ref, wait_ref,
             pw1a, pw1f, pw1t, pw1m, pb1_r, pw2_r, pb2_r,
             tw1a, tw1m, tb1_r, tw2_r, tb2_r,
             mw1a, mw1m, mb1_r, mw2_r, mb2_r,
             y_ref):
        g = pl.program_id(0)
        start = g * BLK
        end = start + BLK
        b1 = bounds_ref[0]
        b2 = bounds_ref[1]
        b3 = bounds_ref[2]
        row = start + lax.broadcasted_iota(jnp.int32, (BLK, 1), 0)

        y_ref[...] = jnp.broadcast_to(wait_ref[...], (BLK, OUT))

        def mlp(parts, w1s, b1v, w2, b2v):
            acc = None
            for x, w in zip(parts, w1s):
                p = jnp.dot(x.astype(jnp.bfloat16), w[...],
                            preferred_element_type=jnp.float32)
                acc = p if acc is None else acc + p
            hh = acc + b1v[...]
            hh = jnp.where(hh >= 0.0, hh, 0.01 * hh)
            return jnp.dot(hh.astype(jnp.bfloat16), w2[...],
                           preferred_element_type=jnp.float32) + b2v[...]

        @pl.when((start < b2) & (end > b1))
        def _():
            y = mlp([a_ref[...], f_ref[...], t_ref[...], m_ref[...]],
                    [pw1a, pw1f, pw1t, pw1m], pb1_r, pw2_r, pb2_r)
            msk = (row >= b1) & (row < b2)
            y_ref[...] = jnp.where(msk, y, y_ref[...])

        @pl.when((start < b3) & (end > b2))
        def _():
            y = mlp([a_ref[...], m_ref[...]], [tw1a, tw1m], tb1_r, tw2_r, tb2_r)
            msk = (row >= b2) & (row < b3)
            y_ref[...] = jnp.where(msk, y, y_ref[...])

        @pl.when(end > b3)
        def _():
            y = mlp([a_ref[...], m_ref[...]], [mw1a, mw1m], mb1_r, mw2_r, mb2_r)
            y_ref[...] = jnp.where(row >= b3, y, y_ref[...])

    def am_map(g, b):
        lo = jnp.minimum(b[0] // BLK, NBLK - 1)
        return jnp.maximum(g, lo), 0

    def ft_map(g, b):
        lo = jnp.minimum(b[0] // BLK, NBLK - 1)
        hi = jnp.clip((b[1] - 1) // BLK, lo, NBLK - 1)
        return jnp.clip(g, lo, hi), 0

    xspec_am = pl.BlockSpec((BLK, C), am_map)
    xspec_ft = pl.BlockSpec((BLK, C), ft_map)
    wfull = pl.BlockSpec((C, H), lambda g, b: (0, 0))
    w2full = pl.BlockSpec((H, OUT), lambda g, b: (0, 0))
    bvec = pl.BlockSpec((1, H), lambda g, b: (0, 0))
    bvec2 = pl.BlockSpec((1, OUT), lambda g, b: (0, 0))

    grid_spec = pltpu.PrefetchScalarGridSpec(
        num_scalar_prefetch=1,
        grid=(NBLK,),
        in_specs=[
            xspec_am, xspec_ft, xspec_ft, xspec_am,
            bvec2,
            wfull, wfull, wfull, wfull, bvec, w2full, bvec2,
            wfull, wfull, bvec, w2full, bvec2,
            wfull, wfull, bvec, w2full, bvec2,
        ],
        out_specs=pl.BlockSpec((BLK, OUT), lambda g, b: (g, 0)),
    )
    return pl.pallas_call(
        body,
        grid_spec=grid_spec,
        out_shape=jax.ShapeDtypeStruct((N, OUT), jnp.float32),
    )(bounds, a, f, t, m, wait_row,
      pw1[0], pw1[1], pw1[2], pw1[3], pb1, pw2, pb2,
      tw1[0], tw1[1], tb1, tw2, tb2,
      mw1[0], mw1[1], mb1, mw2, mb2)


def kernel(action_type, agv_idx, op_from_idx, op_to_idx, machine_idx, cu_seqlens,
           emb_AGV, emb_operation, emb_machine, wait_emb,
           pick_W1, pick_b1, pick_W2, pick_b2,
           trans_W1, trans_b1, trans_W2, trans_b2,
           move_W1, move_b1, move_W2, move_b2):
    at = action_type.astype(jnp.int32)
    order = jnp.argsort(at).astype(jnp.int32)
    sorted_t = jnp.take(at, order)
    bounds = jnp.searchsorted(sorted_t, jnp.arange(1, 4, dtype=jnp.int32)).astype(jnp.int32)

    bounds_rep = jnp.repeat(bounds[:2], 16).astype(jnp.int32)
    a, f, t, m = _sc_gather4(order, agv_idx.astype(jnp.int32),
                             op_from_idx.astype(jnp.int32),
                             op_to_idx.astype(jnp.int32),
                             machine_idx.astype(jnp.int32),
                             emb_AGV, emb_operation, emb_machine, bounds_rep)

    bf = jnp.bfloat16
    pw1 = [pick_W1[i * C:(i + 1) * C].astype(bf) for i in range(4)]
    tw1 = [trans_W1[i * C:(i + 1) * C].astype(bf) for i in range(2)]
    mw1 = [move_W1[i * C:(i + 1) * C].astype(bf) for i in range(2)]

    y_sorted = _tc_mlps(
        a, f, t, m, bounds, wait_emb.reshape(1, OUT),
        pw1, pick_b1.reshape(1, H), pick_W2.astype(bf), pick_b2.reshape(1, OUT),
        tw1, trans_b1.reshape(1, H), trans_W2.astype(bf), trans_b2.reshape(1, OUT),
        mw1, move_b1.reshape(1, H), move_W2.astype(bf), move_b2.reshape(1, OUT))

    return _sc_permute_rows(y_sorted, order)
